# Initial kernel scaffold; baseline (speedup 1.0000x reference)
import functools

import jax
import jax.numpy as jnp
from jax import lax
from jax.experimental import pallas as pl
from jax.experimental.pallas import tpu as pltpu
from jax.experimental.pallas import tpu_sc as plsc

_N = 10000
_E = 320000
_D = 128

_NC = 2
_NS = 16
_NW = _NC * _NS
_CHUNK = 125
_ROWS = _E // _CHUNK
_CPW = _ROWS // _NW
_NPT = _N // _NS

_mesh = plsc.VectorSubcoreMesh(core_axis_name="c", subcore_axis_name="s")


@functools.partial(
    pl.kernel,
    out_type=jax.ShapeDtypeStruct((_NC, _N), jnp.float32),
    mesh=_mesh,
    scratch_types=[
        pltpu.VMEM_SHARED((_N,), jnp.float32),
        pltpu.VMEM((_CPW, _CHUNK), jnp.int32),
        pltpu.VMEM((_CHUNK,), jnp.float32),
    ],
)
def _deg_kernel(dst_hbm, zeros_hbm, ones_hbm, out_hbm, deg_sh, idx_l, ones_l):
    c = lax.axis_index("c")
    s = lax.axis_index("s")
    w = c * _NS + s

    @pl.when(s == 0)
    def _():
        pltpu.sync_copy(zeros_hbm, deg_sh)

    pltpu.sync_copy(ones_hbm, ones_l)
    pltpu.sync_copy(dst_hbm.at[pl.ds(w * _CPW, _CPW)], idx_l)
    plsc.subcore_barrier()

    def body(j, carry):
        pltpu.sync_copy(ones_l, deg_sh.at[idx_l.at[j]], add=True)
        return carry

    lax.fori_loop(0, _CPW, body, 0)
    plsc.subcore_barrier()

    @pl.when(s == 0)
    def _():
        pltpu.sync_copy(deg_sh, out_hbm.at[c])


def _deg(dst_r):
    zeros = jnp.zeros((_N,), jnp.float32)
    ones = jnp.ones((_CHUNK,), jnp.float32)
    return _deg_kernel(dst_r, zeros, ones)


@functools.partial(
    pl.kernel,
    out_type=jax.ShapeDtypeStruct((_NC, _N, _D), jnp.float32),
    mesh=_mesh,
    scratch_types=[
        pltpu.VMEM_SHARED((_N, _D), jnp.float32),
        pltpu.VMEM((_CPW, _CHUNK), jnp.int32),
        pltpu.VMEM((_CPW, _CHUNK), jnp.int32),
        pltpu.VMEM((_CHUNK, _D), jnp.float32),
        pltpu.VMEM((_CHUNK, _D), jnp.float32),
        pltpu.SemaphoreType.DMA,
        pltpu.SemaphoreType.DMA,
    ],
)
def _scatter_kernel(y_hbm, src_hbm, dstidx_hbm, zeros_hbm, out_hbm,
                    acc_sh, src_l, dst_l, buf0, buf1, sem0, sem1):
    c = lax.axis_index("c")
    s = lax.axis_index("s")
    w = c * _NS + s

    pltpu.sync_copy(zeros_hbm.at[pl.ds(s * _NPT, _NPT)],
                    acc_sh.at[pl.ds(s * _NPT, _NPT)])
    pltpu.sync_copy(src_hbm.at[pl.ds(w * _CPW, _CPW)], src_l)
    pltpu.sync_copy(dstidx_hbm.at[pl.ds(w * _CPW, _CPW)], dst_l)
    plsc.subcore_barrier()

    pltpu.async_copy(y_hbm.at[src_l.at[0]], buf0, sem0)

    def body(i, carry):
        j0 = 2 * i
        j1 = 2 * i + 1
        pltpu.async_copy(y_hbm.at[src_l.at[j1]], buf1, sem1)
        pltpu.make_async_copy(y_hbm.at[pl.ds(0, _CHUNK)], buf0, sem0).wait()
        pltpu.sync_copy(buf0, acc_sh.at[dst_l.at[j0]], add=True)

        @pl.when(j1 + 1 < _CPW)
        def _():
            pltpu.async_copy(y_hbm.at[src_l.at[j1 + 1]], buf0, sem0)

        pltpu.make_async_copy(y_hbm.at[pl.ds(0, _CHUNK)], buf1, sem1).wait()
        pltpu.sync_copy(buf1, acc_sh.at[dst_l.at[j1]], add=True)
        return carry

    lax.fori_loop(0, _CPW // 2, body, 0)
    plsc.subcore_barrier()

    pltpu.sync_copy(acc_sh.at[pl.ds(s * _NPT, _NPT)],
                    out_hbm.at[c, pl.ds(s * _NPT, _NPT)])


def _scatter(y, src_r, dst_r):
    zeros = jnp.zeros((_N, _D), jnp.float32)
    return _scatter_kernel(y, src_r, dst_r, zeros)


_BLK = 1024
_GRID = (_N + _BLK - 1) // _BLK


def _mm_body(x_ref, w_ref, degp_ref, y_ref, dis_ref):
    deg = degp_ref[:, 0:1] + degp_ref[:, 1:2] + 1.0
    dis = lax.rsqrt(deg)
    xw = jnp.dot(x_ref[...], w_ref[...], preferred_element_type=jnp.float32)
    y_ref[...] = xw * dis
    dis_ref[...] = dis


def _mm(x, W, degp_t):
    return pl.pallas_call(
        _mm_body,
        grid=(_GRID,),
        in_specs=[
            pl.BlockSpec((_BLK, _D), lambda i: (i, 0)),
            pl.BlockSpec((_D, _D), lambda i: (0, 0)),
            pl.BlockSpec((_BLK, _NC), lambda i: (i, 0)),
        ],
        out_specs=[
            pl.BlockSpec((_BLK, _D), lambda i: (i, 0)),
            pl.BlockSpec((_BLK, 1), lambda i: (i, 0)),
        ],
        out_shape=[
            jax.ShapeDtypeStruct((_N, _D), jnp.float32),
            jax.ShapeDtypeStruct((_N, 1), jnp.float32),
        ],
    )(x, W, degp_t)


def _post1_body(acc_ref, y_ref, dis_ref, b_ref, t_ref, s_ref):
    i = pl.program_id(0)
    a = acc_ref[0] + acc_ref[1] + y_ref[...]
    pre = a * dis_ref[...] + b_ref[...]
    t = jnp.where(pre > 0, pre, 0.01 * pre)
    rows = i * _BLK + lax.broadcasted_iota(jnp.int32, (_BLK, _D), 0)
    t = jnp.where(rows < _N, t, 0.0)
    t_ref[...] = t

    @pl.when(i == 0)
    def _():
        s_ref[...] = jnp.zeros((2, _D), jnp.float32)

    s_ref[0:1, :] += jnp.sum(t, axis=0, keepdims=True)
    s_ref[1:2, :] += jnp.sum(t * t, axis=0, keepdims=True)


def _post1(acc, y, dis, b):
    return pl.pallas_call(
        _post1_body,
        grid=(_GRID,),
        in_specs=[
            pl.BlockSpec((_NC, _BLK, _D), lambda i: (0, i, 0)),
            pl.BlockSpec((_BLK, _D), lambda i: (i, 0)),
            pl.BlockSpec((_BLK, 1), lambda i: (i, 0)),
            pl.BlockSpec((1, _D), lambda i: (0, 0)),
        ],
        out_specs=[
            pl.BlockSpec((_BLK, _D), lambda i: (i, 0)),
            pl.BlockSpec((2, _D), lambda i: (0, 0)),
        ],
        out_shape=[
            jax.ShapeDtypeStruct((_N, _D), jnp.float32),
            jax.ShapeDtypeStruct((2, _D), jnp.float32),
        ],
    )(acc, y, dis, b)


def _post2_body(t_ref, s_ref, g_ref, bt_ref, o_ref):
    inv_n = 1.0 / _N
    mean = s_ref[0:1, :] * inv_n
    var = s_ref[1:2, :] * inv_n - mean * mean
    inv = lax.rsqrt(var + 1e-5)
    o_ref[...] = (t_ref[...] - mean) * (inv * g_ref[...]) + bt_ref[...]


def _post2(t, sums, gamma, beta):
    return pl.pallas_call(
        _post2_body,
        grid=(_GRID,),
        in_specs=[
            pl.BlockSpec((_BLK, _D), lambda i: (i, 0)),
            pl.BlockSpec((2, _D), lambda i: (0, 0)),
            pl.BlockSpec((1, _D), lambda i: (0, 0)),
            pl.BlockSpec((1, _D), lambda i: (0, 0)),
        ],
        out_specs=pl.BlockSpec((_BLK, _D), lambda i: (i, 0)),
        out_shape=jax.ShapeDtypeStruct((_N, _D), jnp.float32),
    )(t, sums, gamma, beta)


def kernel(x, edge_index, W, b, gamma, beta):
    src_r = edge_index[0].reshape(_ROWS, _CHUNK)
    dst_r = edge_index[1].reshape(_ROWS, _CHUNK)

    degp = _deg(dst_r)
    y, dis = _mm(x, W, degp.T)
    acc = _scatter(y, src_r, dst_r)
    t, sums = _post1(acc, y, dis, b.reshape(1, _D))
    return _post2(t, sums, gamma.reshape(1, _D), beta.reshape(1, _D))


# speedup vs baseline: 35.3873x; 35.3873x over previous
import functools

import jax
import jax.numpy as jnp
from jax import lax
from jax.experimental import pallas as pl
from jax.experimental.pallas import tpu as pltpu
from jax.experimental.pallas import tpu_sc as plsc

_N = 10000
_E = 320000
_D = 128

_NC = 2
_NS = 16
_NW = _NC * _NS
_CHUNK = 80
_CPW = _E // (_NW * _CHUNK)
_G = 5
_IB = _CPW // _G

_mesh = plsc.VectorSubcoreMesh(core_axis_name="c", subcore_axis_name="s")


_RB = 640
_RB_LAST = _N - (_NS - 1) * _RB


def _row_split(s, fn):
    @pl.when(s < _NS - 1)
    def _():
        fn(pl.multiple_of(s * _RB, _RB), _RB)

    @pl.when(s == _NS - 1)
    def _():
        fn((_NS - 1) * _RB, _RB_LAST)


@functools.partial(
    pl.kernel,
    out_type=[
        jax.ShapeDtypeStruct((_N,), jnp.float32),
        jax.ShapeDtypeStruct((_N,), jnp.float32),
    ],
    mesh=_mesh,
    scratch_types=[
        pltpu.VMEM_SHARED((_N,), jnp.float32),
        pltpu.VMEM((_G, _IB, _CHUNK), jnp.int32),
        pltpu.VMEM((_CHUNK,), jnp.float32),
    ],
)
def _deg_kernel(dst_hbm, zeros_hbm, ones_hbm, out0, out1, deg_sh, idx_l,
                ones_l):
    c = lax.axis_index("c")
    s = lax.axis_index("s")
    w = c * _NS + s

    @pl.when(s == 0)
    def _():
        pltpu.sync_copy(zeros_hbm, deg_sh)

    pltpu.sync_copy(ones_hbm, ones_l)
    pltpu.sync_copy(dst_hbm.at[pl.ds(w * _G, _G)], idx_l)
    plsc.subcore_barrier()

    def body(j, carry):
        g = j // _IB
        r = j - g * _IB
        pltpu.sync_copy(ones_l, deg_sh.at[idx_l.at[g, r]], add=True)
        return carry

    lax.fori_loop(0, _CPW, body, 0)
    plsc.subcore_barrier()

    @pl.when((s == 0) & (c == 0))
    def _():
        pltpu.sync_copy(deg_sh, out0)

    @pl.when((s == 0) & (c == 1))
    def _():
        pltpu.sync_copy(deg_sh, out1)


def _deg(dst_r):
    zeros = jnp.zeros((_N,), jnp.float32)
    ones = jnp.ones((_CHUNK,), jnp.float32)
    d0, d1 = _deg_kernel(dst_r, zeros, ones)
    return jnp.stack([d0, d1], axis=1)


@functools.partial(
    pl.kernel,
    out_type=[
        jax.ShapeDtypeStruct((_N, _D), jnp.float32),
        jax.ShapeDtypeStruct((_N, _D), jnp.float32),
    ],
    mesh=_mesh,
    scratch_types=[
        pltpu.VMEM_SHARED((_N, _D), jnp.float32),
        pltpu.VMEM((_IB, _CHUNK), jnp.int32),
        pltpu.VMEM((_IB, _CHUNK), jnp.int32),
        pltpu.VMEM((_CHUNK, _D), jnp.float32),
        pltpu.VMEM((_CHUNK, _D), jnp.float32),
        pltpu.SemaphoreType.DMA,
        pltpu.SemaphoreType.DMA,
    ],
)
def _scatter_kernel(y_hbm, src_hbm, dstidx_hbm, zeros_hbm, out0, out1,
                    acc_sh, src_l, dst_l, buf0, buf1, sem0, sem1):
    c = lax.axis_index("c")
    s = lax.axis_index("s")
    w = c * _NS + s

    def zero(off, sz):
        pltpu.sync_copy(zeros_hbm.at[pl.ds(off, sz)],
                        acc_sh.at[pl.ds(off, sz)])

    _row_split(s, zero)
    plsc.subcore_barrier()

    def group(g, carry):
        pltpu.sync_copy(src_hbm.at[w * _G + g], src_l)
        pltpu.sync_copy(dstidx_hbm.at[w * _G + g], dst_l)
        pltpu.async_copy(y_hbm.at[src_l.at[0]], buf0, sem0)

        def body(i, carry2):
            j0 = 2 * i
            j1 = 2 * i + 1
            pltpu.async_copy(y_hbm.at[src_l.at[j1]], buf1, sem1)
            pltpu.make_async_copy(y_hbm.at[pl.ds(0, _CHUNK)], buf0,
                                  sem0).wait()
            pltpu.sync_copy(buf0, acc_sh.at[dst_l.at[j0]], add=True)

            @pl.when(j1 + 1 < _IB)
            def _():
                pltpu.async_copy(y_hbm.at[src_l.at[j1 + 1]], buf0, sem0)

            pltpu.make_async_copy(y_hbm.at[pl.ds(0, _CHUNK)], buf1,
                                  sem1).wait()
            pltpu.sync_copy(buf1, acc_sh.at[dst_l.at[j1]], add=True)
            return carry2

        lax.fori_loop(0, _IB // 2, body, 0)
        if _IB % 2:
            pltpu.make_async_copy(y_hbm.at[pl.ds(0, _CHUNK)], buf0,
                                  sem0).wait()
            pltpu.sync_copy(buf0, acc_sh.at[dst_l.at[_IB - 1]], add=True)
        return carry

    lax.fori_loop(0, _G, group, 0)
    plsc.subcore_barrier()

    def wb(off, sz):
        @pl.when(c == 0)
        def _():
            pltpu.sync_copy(acc_sh.at[pl.ds(off, sz)], out0.at[pl.ds(off, sz)])

        @pl.when(c == 1)
        def _():
            pltpu.sync_copy(acc_sh.at[pl.ds(off, sz)], out1.at[pl.ds(off, sz)])

    _row_split(s, wb)


def _scatter(y, src_r, dst_r):
    zeros = jnp.zeros((_N, _D), jnp.float32)
    return _scatter_kernel(y, src_r, dst_r, zeros)


_BLK = 1024
_GRID = (_N + _BLK - 1) // _BLK


def _mm_body(x_ref, w_ref, degp_ref, y_ref, dis_ref):
    deg = degp_ref[:, 0:1] + degp_ref[:, 1:2] + 1.0
    dis = lax.rsqrt(deg)
    xw = jnp.dot(x_ref[...], w_ref[...], preferred_element_type=jnp.float32)
    y_ref[...] = xw * dis
    dis_ref[...] = dis


def _mm(x, W, degp_t):
    return pl.pallas_call(
        _mm_body,
        grid=(_GRID,),
        in_specs=[
            pl.BlockSpec((_BLK, _D), lambda i: (i, 0)),
            pl.BlockSpec((_D, _D), lambda i: (0, 0)),
            pl.BlockSpec((_BLK, _NC), lambda i: (i, 0)),
        ],
        out_specs=[
            pl.BlockSpec((_BLK, _D), lambda i: (i, 0)),
            pl.BlockSpec((_BLK, 1), lambda i: (i, 0)),
        ],
        out_shape=[
            jax.ShapeDtypeStruct((_N, _D), jnp.float32),
            jax.ShapeDtypeStruct((_N, 1), jnp.float32),
        ],
    )(x, W, degp_t)


def _post1_body(acc0_ref, acc1_ref, y_ref, dis_ref, b_ref, t_ref, s_ref):
    i = pl.program_id(0)
    a = acc0_ref[...] + acc1_ref[...] + y_ref[...]
    pre = a * dis_ref[...] + b_ref[...]
    t = jnp.where(pre > 0, pre, 0.01 * pre)
    rows = i * _BLK + lax.broadcasted_iota(jnp.int32, (_BLK, _D), 0)
    t = jnp.where(rows < _N, t, 0.0)
    t_ref[...] = t

    @pl.when(i == 0)
    def _():
        s_ref[...] = jnp.zeros((2, _D), jnp.float32)

    s_ref[0:1, :] += jnp.sum(t, axis=0, keepdims=True)
    s_ref[1:2, :] += jnp.sum(t * t, axis=0, keepdims=True)


def _post1(acc0, acc1, y, dis, b):
    return pl.pallas_call(
        _post1_body,
        grid=(_GRID,),
        in_specs=[
            pl.BlockSpec((_BLK, _D), lambda i: (i, 0)),
            pl.BlockSpec((_BLK, _D), lambda i: (i, 0)),
            pl.BlockSpec((_BLK, _D), lambda i: (i, 0)),
            pl.BlockSpec((_BLK, 1), lambda i: (i, 0)),
            pl.BlockSpec((1, _D), lambda i: (0, 0)),
        ],
        out_specs=[
            pl.BlockSpec((_BLK, _D), lambda i: (i, 0)),
            pl.BlockSpec((2, _D), lambda i: (0, 0)),
        ],
        out_shape=[
            jax.ShapeDtypeStruct((_N, _D), jnp.float32),
            jax.ShapeDtypeStruct((2, _D), jnp.float32),
        ],
    )(acc0, acc1, y, dis, b)


def _post2_body(t_ref, s_ref, g_ref, bt_ref, o_ref):
    inv_n = 1.0 / _N
    mean = s_ref[0:1, :] * inv_n
    var = s_ref[1:2, :] * inv_n - mean * mean
    inv = lax.rsqrt(var + 1e-5)
    o_ref[...] = (t_ref[...] - mean) * (inv * g_ref[...]) + bt_ref[...]


def _post2(t, sums, gamma, beta):
    return pl.pallas_call(
        _post2_body,
        grid=(_GRID,),
        in_specs=[
            pl.BlockSpec((_BLK, _D), lambda i: (i, 0)),
            pl.BlockSpec((2, _D), lambda i: (0, 0)),
            pl.BlockSpec((1, _D), lambda i: (0, 0)),
            pl.BlockSpec((1, _D), lambda i: (0, 0)),
        ],
        out_specs=pl.BlockSpec((_BLK, _D), lambda i: (i, 0)),
        out_shape=jax.ShapeDtypeStruct((_N, _D), jnp.float32),
    )(t, sums, gamma, beta)


def kernel(x, edge_index, W, b, gamma, beta):
    src_r = edge_index[0].reshape(_NW * _G, _IB, _CHUNK)
    dst_r = edge_index[1].reshape(_NW * _G, _IB, _CHUNK)

    degp = _deg(dst_r)
    y, dis = _mm(x, W, degp)
    acc0, acc1 = _scatter(y, src_r, dst_r)
    t, sums = _post1(acc0, acc1, y, dis, b.reshape(1, _D))
    return _post2(t, sums, gamma.reshape(1, _D), beta.reshape(1, _D))


# speedup vs baseline: 37.7516x; 1.0668x over previous
import functools

import jax
import jax.numpy as jnp
from jax import lax
from jax.experimental import pallas as pl
from jax.experimental.pallas import tpu as pltpu
from jax.experimental.pallas import tpu_sc as plsc

_N = 10000
_E = 320000
_D = 128

_NC = 2
_NS = 16
_NW = _NC * _NS
_CHUNK = 80
_CPW = _E // (_NW * _CHUNK)
_G = 5
_IB = _CPW // _G

_mesh = plsc.VectorSubcoreMesh(core_axis_name="c", subcore_axis_name="s")


_RB = 640
_RB_LAST = _N - (_NS - 1) * _RB


def _row_split(s, fn):
    @pl.when(s < _NS - 1)
    def _():
        fn(pl.multiple_of(s * _RB, _RB), _RB)

    @pl.when(s == _NS - 1)
    def _():
        fn((_NS - 1) * _RB, _RB_LAST)


@functools.partial(
    pl.kernel,
    out_type=[
        jax.ShapeDtypeStruct((_N,), jnp.float32),
        jax.ShapeDtypeStruct((_N,), jnp.float32),
    ],
    mesh=_mesh,
    scratch_types=[
        pltpu.VMEM_SHARED((_N,), jnp.float32),
        pltpu.VMEM((_G, _IB, _CHUNK), jnp.int32),
        pltpu.VMEM((_CHUNK,), jnp.float32),
        pltpu.SemaphoreType.DMA,
    ],
)
def _deg_kernel(dst_hbm, zeros_hbm, ones_hbm, out0, out1, deg_sh, idx_l,
                ones_l, sem):
    c = lax.axis_index("c")
    s = lax.axis_index("s")
    w = c * _NS + s

    @pl.when(s == 0)
    def _():
        pltpu.sync_copy(zeros_hbm, deg_sh)

    pltpu.sync_copy(ones_hbm, ones_l)
    pltpu.sync_copy(dst_hbm.at[pl.ds(w * _G, _G)], idx_l)
    plsc.subcore_barrier()

    def fire(j, carry):
        g = j // _IB
        r = j - g * _IB
        pltpu.async_copy(ones_l, deg_sh.at[idx_l.at[g, r]], sem, add=True)
        return carry

    lax.fori_loop(0, _CPW, fire, 0)

    def drain(j, carry):
        pltpu.make_async_copy(ones_l, deg_sh.at[pl.ds(0, _CHUNK)],
                              sem).wait()
        return carry

    lax.fori_loop(0, _CPW, drain, 0)
    plsc.subcore_barrier()

    @pl.when((s == 0) & (c == 0))
    def _():
        pltpu.sync_copy(deg_sh, out0)

    @pl.when((s == 0) & (c == 1))
    def _():
        pltpu.sync_copy(deg_sh, out1)


def _deg(dst_r):
    zeros = jnp.zeros((_N,), jnp.float32)
    ones = jnp.ones((_CHUNK,), jnp.float32)
    d0, d1 = _deg_kernel(dst_r, zeros, ones)
    return jnp.stack([d0, d1], axis=1)


@functools.partial(
    pl.kernel,
    out_type=[
        jax.ShapeDtypeStruct((_N, _D), jnp.float32),
        jax.ShapeDtypeStruct((_N, _D), jnp.float32),
    ],
    mesh=_mesh,
    scratch_types=[
        pltpu.VMEM_SHARED((_N, _D), jnp.float32),
        pltpu.VMEM((_IB, _CHUNK), jnp.int32),
        pltpu.VMEM((_IB, _CHUNK), jnp.int32),
        pltpu.VMEM((_CHUNK, _D), jnp.float32),
        pltpu.VMEM((_CHUNK, _D), jnp.float32),
        pltpu.SemaphoreType.DMA,
        pltpu.SemaphoreType.DMA,
    ],
)
def _scatter_kernel(y_hbm, src_hbm, dstidx_hbm, zeros_hbm, out0, out1,
                    acc_sh, src_l, dst_l, buf0, buf1, sem0, sem1):
    c = lax.axis_index("c")
    s = lax.axis_index("s")
    w = c * _NS + s

    def zero(off, sz):
        pltpu.sync_copy(zeros_hbm.at[pl.ds(off, sz)],
                        acc_sh.at[pl.ds(off, sz)])

    _row_split(s, zero)
    plsc.subcore_barrier()

    def group(g, carry):
        pltpu.sync_copy(src_hbm.at[w * _G + g], src_l)
        pltpu.sync_copy(dstidx_hbm.at[w * _G + g], dst_l)
        pltpu.async_copy(y_hbm.at[src_l.at[0]], buf0, sem0)

        def body(i, carry2):
            j0 = 2 * i
            j1 = 2 * i + 1
            pltpu.async_copy(y_hbm.at[src_l.at[j1]], buf1, sem1)
            pltpu.make_async_copy(y_hbm.at[pl.ds(0, _CHUNK)], buf0,
                                  sem0).wait()
            pltpu.sync_copy(buf0, acc_sh.at[dst_l.at[j0]], add=True)

            @pl.when(j1 + 1 < _IB)
            def _():
                pltpu.async_copy(y_hbm.at[src_l.at[j1 + 1]], buf0, sem0)

            pltpu.make_async_copy(y_hbm.at[pl.ds(0, _CHUNK)], buf1,
                                  sem1).wait()
            pltpu.sync_copy(buf1, acc_sh.at[dst_l.at[j1]], add=True)
            return carry2

        lax.fori_loop(0, _IB // 2, body, 0)
        if _IB % 2:
            pltpu.make_async_copy(y_hbm.at[pl.ds(0, _CHUNK)], buf0,
                                  sem0).wait()
            pltpu.sync_copy(buf0, acc_sh.at[dst_l.at[_IB - 1]], add=True)
        return carry

    lax.fori_loop(0, _G, group, 0)
    plsc.subcore_barrier()

    def wb(off, sz):
        @pl.when(c == 0)
        def _():
            pltpu.sync_copy(acc_sh.at[pl.ds(off, sz)], out0.at[pl.ds(off, sz)])

        @pl.when(c == 1)
        def _():
            pltpu.sync_copy(acc_sh.at[pl.ds(off, sz)], out1.at[pl.ds(off, sz)])

    _row_split(s, wb)


def _scatter(y, src_r, dst_r):
    zeros = jnp.zeros((_N, _D), jnp.float32)
    return _scatter_kernel(y, src_r, dst_r, zeros)


_BLK = 1024
_GRID = (_N + _BLK - 1) // _BLK


def _mm_body(x_ref, w_ref, degp_ref, y_ref, dis_ref):
    deg = degp_ref[:, 0:1] + degp_ref[:, 1:2] + 1.0
    dis = lax.rsqrt(deg)
    xw = jnp.dot(x_ref[...], w_ref[...], preferred_element_type=jnp.float32)
    y_ref[...] = xw * dis
    dis_ref[...] = dis


def _mm(x, W, degp_t):
    return pl.pallas_call(
        _mm_body,
        grid=(_GRID,),
        in_specs=[
            pl.BlockSpec((_BLK, _D), lambda i: (i, 0)),
            pl.BlockSpec((_D, _D), lambda i: (0, 0)),
            pl.BlockSpec((_BLK, _NC), lambda i: (i, 0)),
        ],
        out_specs=[
            pl.BlockSpec((_BLK, _D), lambda i: (i, 0)),
            pl.BlockSpec((_BLK, 1), lambda i: (i, 0)),
        ],
        out_shape=[
            jax.ShapeDtypeStruct((_N, _D), jnp.float32),
            jax.ShapeDtypeStruct((_N, 1), jnp.float32),
        ],
    )(x, W, degp_t)


def _post_body(acc0_ref, acc1_ref, y_ref, dis_ref, b_ref, g_ref, bt_ref,
               o_ref, t_sc, s_sc):
    i = pl.program_id(0)

    @pl.when(i == 0)
    def _():
        s_sc[...] = jnp.zeros((2, _D), jnp.float32)

    @pl.when(i < _GRID)
    def _():
        a = acc0_ref[...] + acc1_ref[...] + y_ref[...]
        pre = a * dis_ref[...] + b_ref[...]
        t = jnp.where(pre > 0, pre, 0.01 * pre)
        rows = i * _BLK + lax.broadcasted_iota(jnp.int32, (_BLK, _D), 0)
        t = jnp.where(rows < _N, t, 0.0)
        t_sc[pl.ds(pl.multiple_of(i * _BLK, _BLK), _BLK), :] = t
        s_sc[0:1, :] += jnp.sum(t, axis=0, keepdims=True)
        s_sc[1:2, :] += jnp.sum(t * t, axis=0, keepdims=True)

    @pl.when(i >= _GRID)
    def _():
        k = i - _GRID
        inv_n = 1.0 / _N
        mean = s_sc[0:1, :] * inv_n
        var = s_sc[1:2, :] * inv_n - mean * mean
        inv = lax.rsqrt(var + 1e-5)
        t = t_sc[pl.ds(pl.multiple_of(k * _BLK, _BLK), _BLK), :]
        o_ref[...] = (t - mean) * (inv * g_ref[...]) + bt_ref[...]


def _post(acc0, acc1, y, dis, b, gamma, beta):
    ng = _GRID - 1
    return pl.pallas_call(
        _post_body,
        grid=(2 * _GRID,),
        in_specs=[
            pl.BlockSpec((_BLK, _D), lambda i: (jnp.minimum(i, ng), 0)),
            pl.BlockSpec((_BLK, _D), lambda i: (jnp.minimum(i, ng), 0)),
            pl.BlockSpec((_BLK, _D), lambda i: (jnp.minimum(i, ng), 0)),
            pl.BlockSpec((_BLK, 1), lambda i: (jnp.minimum(i, ng), 0)),
            pl.BlockSpec((1, _D), lambda i: (0, 0)),
            pl.BlockSpec((1, _D), lambda i: (0, 0)),
            pl.BlockSpec((1, _D), lambda i: (0, 0)),
        ],
        out_specs=pl.BlockSpec(
            (_BLK, _D), lambda i: (jnp.maximum(i - _GRID, 0), 0)),
        out_shape=jax.ShapeDtypeStruct((_N, _D), jnp.float32),
        scratch_shapes=[
            pltpu.VMEM((_GRID * _BLK, _D), jnp.float32),
            pltpu.VMEM((2, _D), jnp.float32),
        ],
    )(acc0, acc1, y, dis, b, gamma, beta)


def kernel(x, edge_index, W, b, gamma, beta):
    src_r = edge_index[0].reshape(_NW * _G, _IB, _CHUNK)
    dst_r = edge_index[1].reshape(_NW * _G, _IB, _CHUNK)

    degp = _deg(dst_r)
    y, dis = _mm(x, W, degp)
    acc0, acc1 = _scatter(y, src_r, dst_r)
    return _post(acc0, acc1, y, dis, b.reshape(1, _D),
                 gamma.reshape(1, _D), beta.reshape(1, _D))


# speedup vs baseline: 41.2637x; 1.0930x over previous
import functools

import jax
import jax.numpy as jnp
from jax import lax
from jax.experimental import pallas as pl
from jax.experimental.pallas import tpu as pltpu
from jax.experimental.pallas import tpu_sc as plsc

_N = 10000
_E = 320000
_D = 128

_NC = 2
_NS = 16
_NW = _NC * _NS
_CHUNK = 80
_CPW = _E // (_NW * _CHUNK)
_G = 5
_IB = _CPW // _G

_mesh = plsc.VectorSubcoreMesh(core_axis_name="c", subcore_axis_name="s")


_RB = 640
_RB_LAST = _N - (_NS - 1) * _RB


def _row_split(s, fn):
    @pl.when(s < _NS - 1)
    def _():
        fn(pl.multiple_of(s * _RB, _RB), _RB)

    @pl.when(s == _NS - 1)
    def _():
        fn((_NS - 1) * _RB, _RB_LAST)


@functools.partial(
    pl.kernel,
    out_type=[
        jax.ShapeDtypeStruct((_N,), jnp.float32),
        jax.ShapeDtypeStruct((_N,), jnp.float32),
    ],
    mesh=_mesh,
    scratch_types=[
        pltpu.VMEM_SHARED((_N,), jnp.float32),
        pltpu.VMEM((_G, _IB, _CHUNK), jnp.int32),
        pltpu.VMEM((_CHUNK,), jnp.float32),
        pltpu.SemaphoreType.DMA,
    ],
)
def _deg_kernel(dst_hbm, zeros_hbm, ones_hbm, out0, out1, deg_sh, idx_l,
                ones_l, sem):
    c = lax.axis_index("c")
    s = lax.axis_index("s")
    w = c * _NS + s

    @pl.when(s == 0)
    def _():
        pltpu.sync_copy(zeros_hbm, deg_sh)

    pltpu.sync_copy(ones_hbm, ones_l)
    pltpu.sync_copy(dst_hbm.at[pl.ds(w * _G, _G)], idx_l)
    plsc.subcore_barrier()

    def fire(j, carry):
        g = j // _IB
        r = j - g * _IB
        pltpu.async_copy(ones_l, deg_sh.at[idx_l.at[g, r]], sem, add=True)
        return carry

    lax.fori_loop(0, _CPW, fire, 0)

    def drain(j, carry):
        pltpu.make_async_copy(ones_l, deg_sh.at[pl.ds(0, _CHUNK)],
                              sem).wait()
        return carry

    lax.fori_loop(0, _CPW, drain, 0)
    plsc.subcore_barrier()

    @pl.when((s == 0) & (c == 0))
    def _():
        pltpu.sync_copy(deg_sh, out0)

    @pl.when((s == 0) & (c == 1))
    def _():
        pltpu.sync_copy(deg_sh, out1)


def _deg(dst_r):
    zeros = jnp.zeros((_N,), jnp.float32)
    ones = jnp.ones((_CHUNK,), jnp.float32)
    d0, d1 = _deg_kernel(dst_r, zeros, ones)
    return jnp.stack([d0, d1], axis=1)


@functools.partial(
    pl.kernel,
    out_type=[
        jax.ShapeDtypeStruct((_N, _D), jnp.float32),
        jax.ShapeDtypeStruct((_N, _D), jnp.float32),
    ],
    mesh=_mesh,
    scratch_types=[
        pltpu.VMEM_SHARED((_N, _D), jnp.float32),
        pltpu.VMEM((_IB, _CHUNK), jnp.int32),
        pltpu.VMEM((_IB, _CHUNK), jnp.int32),
        pltpu.VMEM((_CHUNK, _D), jnp.float32),
        pltpu.VMEM((_CHUNK, _D), jnp.float32),
        pltpu.VMEM((_CHUNK, _D), jnp.float32),
        pltpu.SemaphoreType.DMA,
        pltpu.SemaphoreType.DMA,
        pltpu.SemaphoreType.DMA,
        pltpu.SemaphoreType.DMA,
        pltpu.SemaphoreType.DMA,
        pltpu.SemaphoreType.DMA,
    ],
)
def _scatter_kernel(y_hbm, src_hbm, dstidx_hbm, zeros_hbm, out0, out1,
                    acc_sh, src_l, dst_l, buf0, buf1, buf2,
                    sg0, sg1, sg2, ss0, ss1, ss2):
    c = lax.axis_index("c")
    s = lax.axis_index("s")
    w = c * _NS + s

    def zero(off, sz):
        pltpu.sync_copy(zeros_hbm.at[pl.ds(off, sz)],
                        acc_sh.at[pl.ds(off, sz)])

    _row_split(s, zero)
    plsc.subcore_barrier()

    def st_g(j, buf, sem):
        pltpu.async_copy(y_hbm.at[src_l.at[j]], buf, sem)

    def st_s(j, buf, sem):
        pltpu.async_copy(buf, acc_sh.at[dst_l.at[j]], sem, add=True)

    def w_g(buf, sem):
        pltpu.make_async_copy(y_hbm.at[pl.ds(0, _CHUNK)], buf, sem).wait()

    def w_s(buf, sem):
        pltpu.make_async_copy(buf, acc_sh.at[pl.ds(0, _CHUNK)], sem).wait()

    def group(g, carry):
        pltpu.sync_copy(src_hbm.at[w * _G + g], src_l)
        pltpu.sync_copy(dstidx_hbm.at[w * _G + g], dst_l)
        st_g(0, buf0, sg0)
        st_g(1, buf1, sg1)
        w_g(buf0, sg0); st_s(0, buf0, ss0); st_g(2, buf2, sg2)

        def tri(i, carry2):
            j = 3 * i
            w_g(buf1, sg1); st_s(j + 1, buf1, ss1)
            w_s(buf0, ss0); st_g(j + 3, buf0, sg0)
            w_g(buf2, sg2); st_s(j + 2, buf2, ss2)
            w_s(buf1, ss1); st_g(j + 4, buf1, sg1)
            w_g(buf0, sg0); st_s(j + 3, buf0, ss0)
            w_s(buf2, ss2); st_g(j + 5, buf2, sg2)
            return carry2

        lax.fori_loop(0, 7, tri, 0)
        w_g(buf1, sg1); st_s(22, buf1, ss1)
        w_s(buf0, ss0); st_g(24, buf0, sg0)
        w_g(buf2, sg2); st_s(23, buf2, ss2)
        w_g(buf0, sg0); st_s(24, buf0, ss0)
        w_s(buf1, ss1); w_s(buf2, ss2); w_s(buf0, ss0)
        return carry

    lax.fori_loop(0, _G, group, 0)
    plsc.subcore_barrier()

    def wb(off, sz):
        @pl.when(c == 0)
        def _():
            pltpu.sync_copy(acc_sh.at[pl.ds(off, sz)], out0.at[pl.ds(off, sz)])

        @pl.when(c == 1)
        def _():
            pltpu.sync_copy(acc_sh.at[pl.ds(off, sz)], out1.at[pl.ds(off, sz)])

    _row_split(s, wb)


def _scatter(y, src_r, dst_r):
    zeros = jnp.zeros((_N, _D), jnp.float32)
    return _scatter_kernel(y, src_r, dst_r, zeros)


_BLK = 1024
_GRID = (_N + _BLK - 1) // _BLK


def _mm_body(x_ref, w_ref, degp_ref, y_ref, dis_ref):
    deg = degp_ref[:, 0:1] + degp_ref[:, 1:2] + 1.0
    dis = lax.rsqrt(deg)
    xw = jnp.dot(x_ref[...], w_ref[...], preferred_element_type=jnp.float32)
    y_ref[...] = xw * dis
    dis_ref[...] = dis


def _mm(x, W, degp_t):
    return pl.pallas_call(
        _mm_body,
        grid=(_GRID,),
        in_specs=[
            pl.BlockSpec((_BLK, _D), lambda i: (i, 0)),
            pl.BlockSpec((_D, _D), lambda i: (0, 0)),
            pl.BlockSpec((_BLK, _NC), lambda i: (i, 0)),
        ],
        out_specs=[
            pl.BlockSpec((_BLK, _D), lambda i: (i, 0)),
            pl.BlockSpec((_BLK, 1), lambda i: (i, 0)),
        ],
        out_shape=[
            jax.ShapeDtypeStruct((_N, _D), jnp.float32),
            jax.ShapeDtypeStruct((_N, 1), jnp.float32),
        ],
    )(x, W, degp_t)


def _post_body(acc0_ref, acc1_ref, y_ref, dis_ref, b_ref, g_ref, bt_ref,
               o_ref, t_sc, s_sc):
    i = pl.program_id(0)

    @pl.when(i == 0)
    def _():
        s_sc[...] = jnp.zeros((2, _D), jnp.float32)

    @pl.when(i < _GRID)
    def _():
        a = acc0_ref[...] + acc1_ref[...] + y_ref[...]
        pre = a * dis_ref[...] + b_ref[...]
        t = jnp.where(pre > 0, pre, 0.01 * pre)
        rows = i * _BLK + lax.broadcasted_iota(jnp.int32, (_BLK, _D), 0)
        t = jnp.where(rows < _N, t, 0.0)
        t_sc[pl.ds(pl.multiple_of(i * _BLK, _BLK), _BLK), :] = t
        s_sc[0:1, :] += jnp.sum(t, axis=0, keepdims=True)
        s_sc[1:2, :] += jnp.sum(t * t, axis=0, keepdims=True)

    @pl.when(i >= _GRID)
    def _():
        k = i - _GRID
        inv_n = 1.0 / _N
        mean = s_sc[0:1, :] * inv_n
        var = s_sc[1:2, :] * inv_n - mean * mean
        inv = lax.rsqrt(var + 1e-5)
        t = t_sc[pl.ds(pl.multiple_of(k * _BLK, _BLK), _BLK), :]
        o_ref[...] = (t - mean) * (inv * g_ref[...]) + bt_ref[...]


def _post(acc0, acc1, y, dis, b, gamma, beta):
    ng = _GRID - 1
    return pl.pallas_call(
        _post_body,
        grid=(2 * _GRID,),
        in_specs=[
            pl.BlockSpec((_BLK, _D), lambda i: (jnp.minimum(i, ng), 0)),
            pl.BlockSpec((_BLK, _D), lambda i: (jnp.minimum(i, ng), 0)),
            pl.BlockSpec((_BLK, _D), lambda i: (jnp.minimum(i, ng), 0)),
            pl.BlockSpec((_BLK, 1), lambda i: (jnp.minimum(i, ng), 0)),
            pl.BlockSpec((1, _D), lambda i: (0, 0)),
            pl.BlockSpec((1, _D), lambda i: (0, 0)),
            pl.BlockSpec((1, _D), lambda i: (0, 0)),
        ],
        out_specs=pl.BlockSpec(
            (_BLK, _D), lambda i: (jnp.maximum(i - _GRID, 0), 0)),
        out_shape=jax.ShapeDtypeStruct((_N, _D), jnp.float32),
        scratch_shapes=[
            pltpu.VMEM((_GRID * _BLK, _D), jnp.float32),
            pltpu.VMEM((2, _D), jnp.float32),
        ],
    )(acc0, acc1, y, dis, b, gamma, beta)


def kernel(x, edge_index, W, b, gamma, beta):
    src_r = edge_index[0].reshape(_NW * _G, _IB, _CHUNK)
    dst_r = edge_index[1].reshape(_NW * _G, _IB, _CHUNK)

    degp = _deg(dst_r)
    y, dis = _mm(x, W, degp)
    acc0, acc1 = _scatter(y, src_r, dst_r)
    return _post(acc0, acc1, y, dis, b.reshape(1, _D),
                 gamma.reshape(1, _D), beta.reshape(1, _D))
